# baseline (speedup 1.0000x reference)
import functools

import jax
import jax.numpy as jnp
from jax import lax
from jax.experimental import pallas as pl
from jax.experimental.pallas import tpu as pltpu
from jax.experimental.pallas import tpu_sc as plsc

_EMBED_DIM = 32
_CHUNK = 128


@functools.lru_cache(maxsize=None)
def _make_gather(batch: int, vocab: int, dim: int):
    info = plsc.get_sparse_core_info()
    num_cores, num_subcores = info.num_cores, info.num_subcores
    num_workers = num_cores * num_subcores
    b_per_w = batch // num_workers
    chunk = min(_CHUNK, b_per_w)
    n_chunks = b_per_w // chunk
    mesh = plsc.VectorSubcoreMesh(core_axis_name="c", subcore_axis_name="s")

    @functools.partial(
        pl.kernel,
        mesh=mesh,
        compiler_params=pltpu.CompilerParams(use_tc_tiling_on_sc=False),
        out_type=jax.ShapeDtypeStruct((batch, dim), jnp.float32),
        scratch_types=[
            pltpu.VMEM((chunk,), jnp.int32),
            pltpu.VMEM((chunk, dim), jnp.float32),
            pltpu.SemaphoreType.DMA,
        ],
    )
    def gather_kernel(idx_hbm, table_hbm, out_hbm, idx_v, rows_v, sem):
        wid = lax.axis_index("s") * num_cores + lax.axis_index("c")
        base = wid * b_per_w

        def body(c, carry):
            off = pl.multiple_of(base + c * chunk, 8)
            pltpu.sync_copy(idx_hbm.at[pl.ds(off, chunk)], idx_v)
            pltpu.async_copy(table_hbm.at[idx_v], rows_v, sem).wait()
            pltpu.sync_copy(rows_v, out_hbm.at[pl.ds(off, chunk)])
            return carry

        lax.fori_loop(0, n_chunks, body, 0)

    return gather_kernel


def kernel(indices, table):
    batch, seq = indices.shape
    vocab, dim = table.shape
    idx_flat = indices.reshape(-1).astype(jnp.int32)
    out = _make_gather(batch * seq, vocab, dim)(idx_flat, table)
    return out.reshape(batch, seq, dim)


# speedup vs baseline: 1.0705x; 1.0705x over previous
import functools

import jax
import jax.numpy as jnp
from jax import lax
from jax.experimental import pallas as pl
from jax.experimental.pallas import tpu as pltpu
from jax.experimental.pallas import tpu_sc as plsc

_EMBED_DIM = 32
_CHUNK = 640


@functools.lru_cache(maxsize=None)
def _make_gather(batch: int, vocab: int, dim: int):
    info = plsc.get_sparse_core_info()
    num_cores, num_subcores = info.num_cores, info.num_subcores
    num_workers = num_cores * num_subcores
    b_per_w = batch // num_workers
    chunk = min(_CHUNK, b_per_w)
    n_chunks = b_per_w // chunk
    mesh = plsc.VectorSubcoreMesh(core_axis_name="c", subcore_axis_name="s")

    @functools.partial(
        pl.kernel,
        mesh=mesh,
        compiler_params=pltpu.CompilerParams(use_tc_tiling_on_sc=False),
        out_type=jax.ShapeDtypeStruct((batch, dim), jnp.float32),
        scratch_types=[
            pltpu.VMEM((b_per_w,), jnp.int32),
            pltpu.VMEM((chunk, dim), jnp.float32),
            pltpu.VMEM((chunk, dim), jnp.float32),
            pltpu.SemaphoreType.DMA,
            pltpu.SemaphoreType.DMA,
        ],
    )
    def gather_kernel(idx_hbm, table_hbm, out_hbm, idx_v, rows0, rows1, sem0, sem1):
        wid = lax.axis_index("s") * num_cores + lax.axis_index("c")
        base = pl.multiple_of(wid * b_per_w, 8)
        pltpu.sync_copy(idx_hbm.at[pl.ds(base, b_per_w)], idx_v)

        bufs = (rows0, rows1)
        sems = (sem0, sem1)

        def fire(c, b):
            return pltpu.async_copy(
                table_hbm.at[idx_v.at[pl.ds(c * chunk, chunk)]], bufs[b], sems[b]
            )

        handles = [None, None]
        handles[0] = fire(0, 0)
        for c in range(n_chunks):
            b = c & 1
            if c + 1 < n_chunks:
                handles[1 - b] = fire(c + 1, 1 - b)
            handles[b].wait()
            pltpu.sync_copy(
                bufs[b], out_hbm.at[pl.ds(base + c * chunk, chunk)]
            )

    return gather_kernel


def kernel(indices, table):
    batch, seq = indices.shape
    vocab, dim = table.shape
    idx_flat = indices.reshape(-1).astype(jnp.int32)
    out = _make_gather(batch * seq, vocab, dim)(idx_flat, table)
    return out.reshape(batch, seq, dim)
